# baseline (speedup 1.0000x reference)
import functools

import jax
import jax.numpy as jnp
from jax import lax
from jax.experimental import pallas as pl
from jax.experimental.pallas import tpu as pltpu
from jax.experimental.pallas import tpu_sc as plsc

N = 10000
NP = 10240
D = 128
R = 8
E = 320000
NC = 2
NS = 16
L = 16

EPS = E // NS
CH = 2000
NCH = EPS // CH
G = 128
HR = 18
RPC = R // NC
RPT = NP // NS


@functools.cache
def _make_agg_kernel(merged):
    mesh = plsc.VectorSubcoreMesh(
        core_axis_name="c", subcore_axis_name="s", num_cores=NC, num_subcores=NS
    )
    if merged:
        outs = (jax.ShapeDtypeStruct((R, NP, D), jnp.float32),
                jax.ShapeDtypeStruct((R, NP, D), jnp.float32))
    else:
        outs = jax.ShapeDtypeStruct((R, NP, D), jnp.float32)
    return functools.partial(
        pl.kernel,
        out_type=outs,
        mesh=mesh,
        compiler_params=pltpu.CompilerParams(needs_layout_passes=False),
        scratch_types=[
            pltpu.VMEM((CH,), jnp.int32),
            pltpu.VMEM((CH,), jnp.int32),
            pltpu.VMEM((CH,), jnp.int32),
            pltpu.VMEM((2 * HR, G), jnp.int32),
            pltpu.VMEM((2 * HR, G), jnp.int32),
            pltpu.VMEM((2, G, D), jnp.float32),
            pltpu.VMEM_SHARED((NP, D), jnp.float32),
            pltpu.SemaphoreType.DMA,
            pltpu.SemaphoreType.DMA,
            pltpu.SemaphoreType.DMA,
        ],
    )(functools.partial(_agg_body, merged))


def _one_pass(gather, rr, c, s, lanes, tr_s, x_hbm, src_hbm, dst_hbm, et_hbm,
              zrows_hbm, out_hbm, srcch, dstch, etch, srcc, dstc, rows2,
              acc_sp, gsem, csem, ssem):
    r = c * RPC + rr
    if gather:
        tr_d = (lanes * 97 + s * 640 + rr * 320) % NP
    else:
        tr_d = N + (lanes * 11 + s * 17 + rr * 41) % (NP - N)

    pltpu.sync_copy(zrows_hbm, acc_sp.at[pl.ds(s * RPT, RPT)])
    plsc.subcore_barrier()

    def _stage(q2):
        base2 = s * EPS + q2 * CH
        if gather:
            pltpu.async_copy(src_hbm.at[pl.ds(base2, CH)], srcch, csem)
        pltpu.async_copy(dst_hbm.at[pl.ds(base2, CH)], dstch, csem)
        pltpu.async_copy(et_hbm.at[pl.ds(base2, CH)], etch, csem)

    def _stage_wait():
        if gather:
            pltpu.make_async_copy(
                src_hbm.at[pl.ds(0, CH)], srcch, csem).wait()
        pltpu.make_async_copy(dst_hbm.at[pl.ds(0, CH)], dstch, csem).wait()
        pltpu.make_async_copy(et_hbm.at[pl.ds(0, CH)], etch, csem).wait()

    _stage(0)

    def chunk(q, carry):
        t0, prow0 = carry
        h = q & 1
        rowbase = h * HR
        _stage_wait()

        def cl(k, off):
            o = k * L
            dv = dstch[pl.ds(o, L)]
            tv = etch[pl.ds(o, L)]
            m = tv == r
            mi = jnp.where(m, 1, 0)
            cum = jnp.cumsum(mi)
            pos = off + cum - mi
            prw = rowbase + (pos >> 7)
            pcl = pos & (G - 1)
            if gather:
                sv = srcch[pl.ds(o, L)]
                plsc.store_scatter(srcc, [prw, pcl], sv, mask=m)
            plsc.store_scatter(dstc, [prw, pcl], dv, mask=m)
            return off + cum[15]

        m_tot = lax.fori_loop(0, CH // L, cl, jnp.int32(0))

        @pl.when(q + 1 < NCH)
        def _():
            _stage(q + 1)

        k16 = (m_tot // L) * L
        rem = m_tot - k16
        krow = rowbase + (k16 >> 7)
        kcol = k16 & (G - 1)
        sch = dstc[krow, pl.ds(kcol, L)]
        dstc[krow, pl.ds(kcol, L)] = jnp.where(lanes < rem, sch, tr_d)
        if gather:
            ssh = srcc[krow, pl.ds(kcol, L)]
            srcc[krow, pl.ds(kcol, L)] = jnp.where(lanes < rem, ssh, tr_s)
        for kk in range(2 * (G // L) + 1):
            p = k16 + L + kk * L
            dstc[rowbase + (p >> 7), pl.ds(p & (G - 1), L)] = tr_d
            if gather:
                srcc[rowbase + (p >> 7), pl.ds(p & (G - 1), L)] = tr_s
        nb = jnp.maximum((m_tot + G - 1) // G, 2)

        def bl(j, carry2):
            t, prow = carry2
            row = rowbase + j
            b = t & 1
            if gather:
                @pl.when(t >= 1)
                def _():
                    pltpu.make_async_copy(
                        x_hbm.at[srcc.at[row]], rows2.at[1 - b], gsem).wait()

                @pl.when(t >= 2)
                def _():
                    pltpu.make_async_copy(
                        rows2.at[b], acc_sp.at[dstc.at[row]], ssem).wait()

                @pl.when(t >= 1)
                def _():
                    pltpu.async_copy(
                        rows2.at[1 - b], acc_sp.at[dstc.at[prow]],
                        ssem, add=True)
                pltpu.async_copy(x_hbm.at[srcc.at[row]], rows2.at[b], gsem)
            else:
                @pl.when(t >= 2)
                def _():
                    pltpu.make_async_copy(
                        rows2.at[0], acc_sp.at[dstc.at[row]], ssem).wait()
                pltpu.async_copy(
                    rows2.at[0], acc_sp.at[dstc.at[row]], ssem, add=True)
            return (t + 1, row)

        return lax.fori_loop(0, nb, bl, (t0, prow0))

    t, prow = lax.fori_loop(0, NCH, chunk, (jnp.int32(0), jnp.int32(0)))

    if gather:
        pltpu.make_async_copy(x_hbm.at[srcc.at[0]], rows2.at[0], gsem).wait()
        pltpu.make_async_copy(rows2.at[0], acc_sp.at[dstc.at[0]], ssem).wait()
        pltpu.async_copy(
            rows2.at[(t - 1) & 1], acc_sp.at[dstc.at[prow]], ssem, add=True)
        pltpu.make_async_copy(rows2.at[0], acc_sp.at[dstc.at[0]], ssem).wait()
    else:
        pltpu.make_async_copy(rows2.at[0], acc_sp.at[dstc.at[0]], ssem).wait()
        pltpu.make_async_copy(rows2.at[0], acc_sp.at[dstc.at[0]], ssem).wait()
    plsc.subcore_barrier()
    pltpu.sync_copy(acc_sp.at[pl.ds(s * RPT, RPT)],
                    out_hbm.at[r, pl.ds(s * RPT, RPT)])
    plsc.subcore_barrier()


def _agg_body(merged, x_hbm, src_hbm, dst_hbm, et_hbm, zrows_hbm, *rest):
    if merged:
        agg_hbm, cnt_hbm = rest[0], rest[1]
        scratch = rest[2:]
    else:
        agg_hbm = rest[0]
        scratch = rest[1:]
    rows2 = scratch[5]
    c = lax.axis_index("c")
    s = lax.axis_index("s")
    lanes = lax.iota(jnp.int32, L)
    tr_s = N + (lanes * 7 + s * 29 + c * 13) % (NP - N)
    for rr in range(RPC):
        _one_pass(True, rr, c, s, lanes, tr_s, x_hbm, src_hbm, dst_hbm,
                  et_hbm, zrows_hbm, agg_hbm, *scratch)
    if merged:
        ones16 = jnp.full((L,), 1.0, jnp.float32)

        def ol(i, _):
            rows2[0, i >> 3, pl.ds((i & 7) * L, L)] = ones16
            return 0

        lax.fori_loop(0, G * (D // L), ol, 0)
        for rr in range(RPC):
            _one_pass(False, rr, c, s, lanes, tr_s, x_hbm, src_hbm, dst_hbm,
                      et_hbm, zrows_hbm, cnt_hbm, *scratch)


BLK = 1024


def _scorer_body(relu, x_ref, agg_ref, cnt_ref, root_ref, w_ref, b_ref, o_ref):
    i = pl.program_id(0)
    x = x_ref[...]
    acc = jnp.dot(x, root_ref[...], preferred_element_type=jnp.float32)
    cnt = cnt_ref[...]
    inv = 1.0 / jnp.maximum(cnt, 1.0)
    agg = agg_ref[...]
    w = w_ref[...]
    for r in range(R):
        acc = acc + jnp.dot(agg[r] * inv[r][:, None], w[r],
                            preferred_element_type=jnp.float32)
    acc = acc + b_ref[...]
    rows = i * BLK + lax.broadcasted_iota(jnp.int32, (BLK, 1), 0)
    acc = jnp.where(rows < N, acc, 0.0)
    if relu:
        acc = jnp.maximum(acc, 0.0)
    o_ref[...] = acc


def _scorer(x, agg, cnt, root, w, b2d, relu):
    return pl.pallas_call(
        functools.partial(_scorer_body, relu),
        grid=(NP // BLK,),
        in_specs=[
            pl.BlockSpec((BLK, D), lambda i: (i, 0)),
            pl.BlockSpec((R, BLK, D), lambda i: (0, i, 0)),
            pl.BlockSpec((R, BLK), lambda i: (0, i)),
            pl.BlockSpec((D, D), lambda i: (0, 0)),
            pl.BlockSpec((R, D, D), lambda i: (0, 0, 0)),
            pl.BlockSpec((1, D), lambda i: (0, 0)),
        ],
        out_specs=pl.BlockSpec((BLK, D), lambda i: (i, 0)),
        out_shape=jax.ShapeDtypeStruct((NP, D), jnp.float32),
    )(x, agg, cnt, root, w, b2d)


def kernel(edge_index, edge_type, node_emb, W1, root1, b1, W2, root2, b2):
    src = edge_index[0].astype(jnp.int32)
    dst = edge_index[1].astype(jnp.int32)
    et = edge_type.astype(jnp.int32)
    x0 = jnp.zeros((NP, D), jnp.float32).at[:N].set(node_emb)
    zrows_d = jnp.zeros((RPT, D), jnp.float32)

    agg1, cntagg = _make_agg_kernel(True)(x0, src, dst, et, zrows_d)
    cnt = cntagg[:, :, 0]
    x1 = _scorer(x0, agg1, cnt, root1, W1, b1.reshape(1, D), relu=True)
    agg2 = _make_agg_kernel(False)(x1, src, dst, et, zrows_d)
    out = _scorer(x1, agg2, cnt, root2, W2, b2.reshape(1, D), relu=False)
    return out[:N]


# speedup vs baseline: 1.0344x; 1.0344x over previous
import functools

import jax
import jax.numpy as jnp
from jax import lax
from jax.experimental import pallas as pl
from jax.experimental.pallas import tpu as pltpu
from jax.experimental.pallas import tpu_sc as plsc

N = 10000
NP = 10240
D = 128
R = 8
E = 320000
NC = 2
NS = 16
L = 16

EPS = E // NS
CH = 2000
NCH = EPS // CH
G = 128
HR = 18
RPC = R // NC
RPT = NP // NS


@functools.cache
def _make_agg_kernel(gather):
    mesh = plsc.VectorSubcoreMesh(
        core_axis_name="c", subcore_axis_name="s", num_cores=NC, num_subcores=NS
    )
    return functools.partial(
        pl.kernel,
        out_type=jax.ShapeDtypeStruct((R, NP, D), jnp.float32),
        mesh=mesh,
        compiler_params=pltpu.CompilerParams(needs_layout_passes=False),
        scratch_types=[
            pltpu.VMEM((CH,), jnp.int32),
            pltpu.VMEM((CH,), jnp.int32),
            pltpu.VMEM((CH,), jnp.int32),
            pltpu.VMEM((2 * HR, G), jnp.int32),
            pltpu.VMEM((2 * HR, G), jnp.int32),
            pltpu.VMEM((2, G, D), jnp.float32),
            pltpu.VMEM_SHARED((NP, D), jnp.float32),
            pltpu.SemaphoreType.DMA,
            pltpu.SemaphoreType.DMA,
            pltpu.SemaphoreType.DMA,
        ],
    )(functools.partial(_agg_body, gather))


def _one_pass(gather, rr, c, s, lanes, tr_s, x_hbm, src_hbm, dst_hbm, et_hbm,
              zrows_hbm, out_hbm, srcch, dstch, etch, srcc, dstc, rows2,
              acc_sp, gsem, csem, ssem):
    r = c * RPC + rr
    if gather:
        tr_d = (lanes * 97 + s * 640 + rr * 320) % NP
    else:
        tr_d = N + (lanes * 11 + s * 17 + rr * 41) % (NP - N)

    pltpu.sync_copy(zrows_hbm, acc_sp.at[pl.ds(s * RPT, RPT)])
    plsc.subcore_barrier()

    def _stage(q2):
        base2 = s * EPS + q2 * CH
        if gather:
            pltpu.async_copy(src_hbm.at[pl.ds(base2, CH)], srcch, csem)
        pltpu.async_copy(dst_hbm.at[pl.ds(base2, CH)], dstch, csem)
        pltpu.async_copy(et_hbm.at[pl.ds(base2, CH)], etch, csem)

    def _stage_wait():
        if gather:
            pltpu.make_async_copy(
                src_hbm.at[pl.ds(0, CH)], srcch, csem).wait()
        pltpu.make_async_copy(dst_hbm.at[pl.ds(0, CH)], dstch, csem).wait()
        pltpu.make_async_copy(et_hbm.at[pl.ds(0, CH)], etch, csem).wait()

    _stage(0)

    def chunk(q, carry):
        t0, prow0 = carry
        h = q & 1
        rowbase = h * HR
        _stage_wait()

        def cl(k, off):
            o = k * L
            dv = dstch[pl.ds(o, L)]
            tv = etch[pl.ds(o, L)]
            m = tv == r
            mi = jnp.where(m, 1, 0)
            cum = jnp.cumsum(mi)
            pos = off + cum - mi
            prw = rowbase + (pos >> 7)
            pcl = pos & (G - 1)
            if gather:
                sv = srcch[pl.ds(o, L)]
                plsc.store_scatter(srcc, [prw, pcl], sv, mask=m)
            plsc.store_scatter(dstc, [prw, pcl], dv, mask=m)
            return off + cum[15]

        m_tot = lax.fori_loop(0, CH // L, cl, jnp.int32(0))

        @pl.when(q + 1 < NCH)
        def _():
            _stage(q + 1)

        k16 = (m_tot // L) * L
        rem = m_tot - k16
        krow = rowbase + (k16 >> 7)
        kcol = k16 & (G - 1)
        sch = dstc[krow, pl.ds(kcol, L)]
        dstc[krow, pl.ds(kcol, L)] = jnp.where(lanes < rem, sch, tr_d)
        if gather:
            ssh = srcc[krow, pl.ds(kcol, L)]
            srcc[krow, pl.ds(kcol, L)] = jnp.where(lanes < rem, ssh, tr_s)
        for kk in range(2 * (G // L) + 1):
            p = k16 + L + kk * L
            dstc[rowbase + (p >> 7), pl.ds(p & (G - 1), L)] = tr_d
            if gather:
                srcc[rowbase + (p >> 7), pl.ds(p & (G - 1), L)] = tr_s
        nb = jnp.maximum((m_tot + G - 1) // G, 2)

        def bl(j, carry2):
            t, prow = carry2
            row = rowbase + j
            b = t & 1
            if gather:
                @pl.when(t >= 1)
                def _():
                    pltpu.make_async_copy(
                        x_hbm.at[srcc.at[row]], rows2.at[1 - b], gsem).wait()

                @pl.when(t >= 2)
                def _():
                    pltpu.make_async_copy(
                        rows2.at[b], acc_sp.at[dstc.at[row]], ssem).wait()

                @pl.when(t >= 1)
                def _():
                    pltpu.async_copy(
                        rows2.at[1 - b], acc_sp.at[dstc.at[prow]],
                        ssem, add=True)
                pltpu.async_copy(x_hbm.at[srcc.at[row]], rows2.at[b], gsem)
            else:
                @pl.when(t >= 2)
                def _():
                    pltpu.make_async_copy(
                        rows2.at[0], acc_sp.at[dstc.at[row]], ssem).wait()
                pltpu.async_copy(
                    rows2.at[0], acc_sp.at[dstc.at[row]], ssem, add=True)
            return (t + 1, row)

        return lax.fori_loop(0, nb, bl, (t0, prow0))

    t, prow = lax.fori_loop(0, NCH, chunk, (jnp.int32(0), jnp.int32(0)))

    if gather:
        pltpu.make_async_copy(x_hbm.at[srcc.at[0]], rows2.at[0], gsem).wait()
        pltpu.make_async_copy(rows2.at[0], acc_sp.at[dstc.at[0]], ssem).wait()
        pltpu.async_copy(
            rows2.at[(t - 1) & 1], acc_sp.at[dstc.at[prow]], ssem, add=True)
        pltpu.make_async_copy(rows2.at[0], acc_sp.at[dstc.at[0]], ssem).wait()
    else:
        pltpu.make_async_copy(rows2.at[0], acc_sp.at[dstc.at[0]], ssem).wait()
        pltpu.make_async_copy(rows2.at[0], acc_sp.at[dstc.at[0]], ssem).wait()
    plsc.subcore_barrier()
    pltpu.sync_copy(acc_sp.at[pl.ds(s * RPT, RPT)],
                    out_hbm.at[r, pl.ds(s * RPT, RPT)])
    plsc.subcore_barrier()


def _agg_body(gather, x_hbm, src_hbm, dst_hbm, et_hbm, zrows_hbm, out_hbm,
              *scratch):
    rows2 = scratch[5]
    c = lax.axis_index("c")
    s = lax.axis_index("s")
    lanes = lax.iota(jnp.int32, L)
    tr_s = N + (lanes * 7 + s * 29 + c * 13) % (NP - N)
    if not gather:
        ones16 = jnp.full((L,), 1.0, jnp.float32)

        def ol(i, _):
            rows2[0, i >> 3, pl.ds((i & 7) * L, L)] = ones16
            return 0

        lax.fori_loop(0, G * (D // L), ol, 0)
    for rr in range(RPC):
        _one_pass(gather, rr, c, s, lanes, tr_s, x_hbm, src_hbm, dst_hbm,
                  et_hbm, zrows_hbm, out_hbm, *scratch)


BLK = 1024


def _scorer_body(relu, x_ref, agg_ref, cnt_ref, root_ref, w_ref, b_ref, o_ref):
    i = pl.program_id(0)
    x = x_ref[...]
    acc = jnp.dot(x, root_ref[...], preferred_element_type=jnp.float32)
    cnt = cnt_ref[...]
    inv = 1.0 / jnp.maximum(cnt, 1.0)
    agg = agg_ref[...]
    w = w_ref[...]
    for r in range(R):
        acc = acc + jnp.dot(agg[r] * inv[r][:, None], w[r],
                            preferred_element_type=jnp.float32)
    acc = acc + b_ref[...]
    rows = i * BLK + lax.broadcasted_iota(jnp.int32, (BLK, 1), 0)
    acc = jnp.where(rows < N, acc, 0.0)
    if relu:
        acc = jnp.maximum(acc, 0.0)
    o_ref[...] = acc


def _scorer(x, agg, cnt, root, w, b2d, relu):
    return pl.pallas_call(
        functools.partial(_scorer_body, relu),
        grid=(NP // BLK,),
        in_specs=[
            pl.BlockSpec((BLK, D), lambda i: (i, 0)),
            pl.BlockSpec((R, BLK, D), lambda i: (0, i, 0)),
            pl.BlockSpec((R, BLK), lambda i: (0, i)),
            pl.BlockSpec((D, D), lambda i: (0, 0)),
            pl.BlockSpec((R, D, D), lambda i: (0, 0, 0)),
            pl.BlockSpec((1, D), lambda i: (0, 0)),
        ],
        out_specs=pl.BlockSpec((BLK, D), lambda i: (i, 0)),
        out_shape=jax.ShapeDtypeStruct((NP, D), jnp.float32),
    )(x, agg, cnt, root, w, b2d)


def kernel(edge_index, edge_type, node_emb, W1, root1, b1, W2, root2, b2):
    src = edge_index[0].astype(jnp.int32)
    dst = edge_index[1].astype(jnp.int32)
    et = edge_type.astype(jnp.int32)
    x0 = jnp.zeros((NP, D), jnp.float32).at[:N].set(node_emb)
    zrows_d = jnp.zeros((RPT, D), jnp.float32)

    agg = _make_agg_kernel(True)
    cnt = _make_agg_kernel(False)(x0, src, dst, et, zrows_d)[:, :, 0]
    agg1 = agg(x0, src, dst, et, zrows_d)
    x1 = _scorer(x0, agg1, cnt, root1, W1, b1.reshape(1, D), relu=True)
    agg2 = agg(x1, src, dst, et, zrows_d)
    out = _scorer(x1, agg2, cnt, root2, W2, b2.reshape(1, D), relu=False)
    return out[:N]
